# baseline (speedup 1.0000x reference)
import functools

import jax
import jax.numpy as jnp
from jax.experimental import pallas as pl

N, D, O, E = 2048, 768, 768, 8
BN = 2048


def _moe_kernel(x_ref, wgt_ref, bg_ref, wet_ref, be_ref, out_ref):
    xb = x_ref[...].astype(jnp.bfloat16)
    logits = jax.lax.dot_general(
        xb, wgt_ref[...].astype(jnp.bfloat16), (((1,), (0,)), ((), ())),
        preferred_element_type=jnp.float32,
    ) + bg_ref[...]

    iota = jax.lax.broadcasted_iota(jnp.int32, logits.shape, 1)
    big = jnp.int32(E)
    v0 = jnp.max(logits, axis=-1, keepdims=True)
    e0 = jnp.min(jnp.where(logits == v0, iota, big), axis=-1, keepdims=True)
    masked = jnp.where(iota == e0, -jnp.inf, logits)
    v1 = jnp.max(masked, axis=-1, keepdims=True)
    e1 = jnp.min(jnp.where(masked == v1, iota, big), axis=-1, keepdims=True)

    w0 = 1.0 / (1.0 + jnp.exp(v1 - v0))
    w1 = 1.0 - w0
    w_full = jnp.where(iota == e0, w0, 0.0) + jnp.where(iota == e1, w1, 0.0)

    acc = jax.lax.dot_general(
        w_full, be_ref[...], (((1,), (0,)), ((), ())),
        preferred_element_type=jnp.float32,
        precision=jax.lax.Precision.HIGHEST,
    )

    for e in range(E):
        web = wet_ref[e][...].astype(jnp.bfloat16)
        pe = jax.lax.dot_general(
            xb, web, (((1,), (1,)), ((), ())),
            preferred_element_type=jnp.float32,
        )
        acc = acc + w_full[:, e][:, None] * pe
    out_ref[...] = acc


@jax.jit
def kernel(x, Wg, bg, We, be):
    wgt = Wg.T
    bg2 = bg[None, :]
    grid = (N // BN,)
    return pl.pallas_call(
        _moe_kernel,
        grid=grid,
        in_specs=[
            pl.BlockSpec((BN, D), lambda i: (i, 0)),
            pl.BlockSpec((D, E), lambda i: (0, 0)),
            pl.BlockSpec((1, E), lambda i: (0, 0)),
            pl.BlockSpec((E, O, D), lambda i: (0, 0, 0)),
            pl.BlockSpec((E, O), lambda i: (0, 0)),
        ],
        out_specs=pl.BlockSpec((BN, O), lambda i: (i, 0)),
        out_shape=jax.ShapeDtypeStruct((N, O), jnp.float32),
    )(x, wgt, bg2, We, be)


# speedup vs baseline: 1.0664x; 1.0664x over previous
import functools

import jax
import jax.numpy as jnp
from jax.experimental import pallas as pl

N, D, O, E = 2048, 768, 768, 8
BN = 512


def _moe_kernel(x_ref, wgt_ref, bg_ref, wet_ref, be_ref, out_ref):
    xb = x_ref[...].astype(jnp.bfloat16)
    logits = jax.lax.dot_general(
        xb, wgt_ref[...].astype(jnp.bfloat16), (((1,), (0,)), ((), ())),
        preferred_element_type=jnp.float32,
    ) + bg_ref[...]

    iota = jax.lax.broadcasted_iota(jnp.int32, logits.shape, 1)
    big = jnp.int32(E)
    v0 = jnp.max(logits, axis=-1, keepdims=True)
    e0 = jnp.min(jnp.where(logits == v0, iota, big), axis=-1, keepdims=True)
    masked = jnp.where(iota == e0, -jnp.inf, logits)
    v1 = jnp.max(masked, axis=-1, keepdims=True)
    e1 = jnp.min(jnp.where(masked == v1, iota, big), axis=-1, keepdims=True)

    w0 = 1.0 / (1.0 + jnp.exp(v1 - v0))
    w1 = 1.0 - w0
    w_full = jnp.where(iota == e0, w0, 0.0) + jnp.where(iota == e1, w1, 0.0)

    acc = jax.lax.dot_general(
        w_full, be_ref[...], (((1,), (0,)), ((), ())),
        preferred_element_type=jnp.float32,
        precision=jax.lax.Precision.HIGHEST,
    )

    for e in range(E):
        web = wet_ref[e][...].astype(jnp.bfloat16)
        pe = jax.lax.dot_general(
            xb, web, (((1,), (1,)), ((), ())),
            preferred_element_type=jnp.float32,
        )
        acc = acc + w_full[:, e][:, None] * pe
    out_ref[...] = acc


@jax.jit
def kernel(x, Wg, bg, We, be):
    wgt = Wg.T
    bg2 = bg[None, :]
    grid = (N // BN,)
    return pl.pallas_call(
        _moe_kernel,
        grid=grid,
        in_specs=[
            pl.BlockSpec((BN, D), lambda i: (i, 0)),
            pl.BlockSpec((D, E), lambda i: (0, 0)),
            pl.BlockSpec((1, E), lambda i: (0, 0)),
            pl.BlockSpec((E, O, D), lambda i: (0, 0, 0)),
            pl.BlockSpec((E, O), lambda i: (0, 0)),
        ],
        out_specs=pl.BlockSpec((BN, O), lambda i: (i, 0)),
        out_shape=jax.ShapeDtypeStruct((N, O), jnp.float32),
    )(x, wgt, bg2, We, be)


# speedup vs baseline: 1.0836x; 1.0161x over previous
import functools

import jax
import jax.numpy as jnp
from jax.experimental import pallas as pl
from jax.experimental.pallas import tpu as pltpu

N, D, O, E = 2048, 768, 768, 8
BN = 512


def _moe_kernel(x_ref, wgt_ref, bg_ref, wet_ref, be_ref, out_ref, web_ref):
    @pl.when(pl.program_id(0) == 0)
    def _cast_weights():
        for e in range(E):
            web_ref[e] = wet_ref[e][...].astype(jnp.bfloat16)

    xb = x_ref[...].astype(jnp.bfloat16)
    logits = jax.lax.dot_general(
        xb, wgt_ref[...].astype(jnp.bfloat16), (((1,), (0,)), ((), ())),
        preferred_element_type=jnp.float32,
    ) + bg_ref[...]

    iota = jax.lax.broadcasted_iota(jnp.int32, logits.shape, 1)
    big = jnp.int32(E)
    v0 = jnp.max(logits, axis=-1, keepdims=True)
    e0 = jnp.min(jnp.where(logits == v0, iota, big), axis=-1, keepdims=True)
    masked = jnp.where(iota == e0, -jnp.inf, logits)
    v1 = jnp.max(masked, axis=-1, keepdims=True)
    e1 = jnp.min(jnp.where(masked == v1, iota, big), axis=-1, keepdims=True)

    w0 = 1.0 / (1.0 + jnp.exp(v1 - v0))
    w1 = 1.0 - w0
    w_full = jnp.where(iota == e0, w0, 0.0) + jnp.where(iota == e1, w1, 0.0)
    wb = w_full.astype(jnp.bfloat16)

    acc = jax.lax.dot_general(
        wb, be_ref[...].astype(jnp.bfloat16), (((1,), (0,)), ((), ())),
        preferred_element_type=jnp.float32,
    )

    for e in range(E):
        xs = xb * wb[:, e][:, None]
        acc = acc + jax.lax.dot_general(
            xs, web_ref[e][...], (((1,), (1,)), ((), ())),
            preferred_element_type=jnp.float32,
        )
    out_ref[...] = acc


@jax.jit
def kernel(x, Wg, bg, We, be):
    wgt = Wg.T
    bg2 = bg[None, :]
    grid = (N // BN,)
    return pl.pallas_call(
        _moe_kernel,
        grid=grid,
        in_specs=[
            pl.BlockSpec((BN, D), lambda i: (i, 0)),
            pl.BlockSpec((D, E), lambda i: (0, 0)),
            pl.BlockSpec((1, E), lambda i: (0, 0)),
            pl.BlockSpec((E, O, D), lambda i: (0, 0, 0)),
            pl.BlockSpec((E, O), lambda i: (0, 0)),
        ],
        out_specs=pl.BlockSpec((BN, O), lambda i: (i, 0)),
        out_shape=jax.ShapeDtypeStruct((N, O), jnp.float32),
        scratch_shapes=[pltpu.VMEM((E, O, D), jnp.bfloat16)],
    )(x, wgt, bg2, We, be)


# speedup vs baseline: 1.1834x; 1.0922x over previous
import functools

import jax
import jax.numpy as jnp
from jax.experimental import pallas as pl
from jax.experimental.pallas import tpu as pltpu

N, D, O, E = 2048, 768, 768, 8
BO = 256


def _moe_kernel(x_ref, wgt_ref, bg_ref, wet_ref, be_ref, out_ref,
                xb_ref, wb_ref):
    @pl.when(pl.program_id(0) == 0)
    def _gate():
        xb0 = x_ref[...].astype(jnp.bfloat16)
        xb_ref[...] = xb0
        logits = jax.lax.dot_general(
            xb0, wgt_ref[...].astype(jnp.bfloat16), (((1,), (0,)), ((), ())),
            preferred_element_type=jnp.float32,
        ) + bg_ref[...]

        iota = jax.lax.broadcasted_iota(jnp.int32, logits.shape, 1)
        big = jnp.int32(E)
        v0 = jnp.max(logits, axis=-1, keepdims=True)
        e0 = jnp.min(jnp.where(logits == v0, iota, big), axis=-1,
                     keepdims=True)
        masked = jnp.where(iota == e0, -jnp.inf, logits)
        v1 = jnp.max(masked, axis=-1, keepdims=True)
        e1 = jnp.min(jnp.where(masked == v1, iota, big), axis=-1,
                     keepdims=True)
        w0 = 1.0 / (1.0 + jnp.exp(v1 - v0))
        w1 = 1.0 - w0
        w_full = (jnp.where(iota == e0, w0, 0.0)
                  + jnp.where(iota == e1, w1, 0.0))
        wb_ref[...] = w_full.astype(jnp.bfloat16)

    xb = xb_ref[...]
    wb = wb_ref[...]

    acc = jax.lax.dot_general(
        wb, be_ref[...].astype(jnp.bfloat16), (((1,), (0,)), ((), ())),
        preferred_element_type=jnp.float32,
    )

    for e in range(E):
        xs = xb * wb[:, e][:, None]
        acc = acc + jax.lax.dot_general(
            xs, wet_ref[e][...].astype(jnp.bfloat16), (((1,), (1,)), ((), ())),
            preferred_element_type=jnp.float32,
        )
    out_ref[...] = acc


@jax.jit
def kernel(x, Wg, bg, We, be):
    wgt = Wg.T
    bg2 = bg[None, :]
    grid = (O // BO,)
    return pl.pallas_call(
        _moe_kernel,
        grid=grid,
        in_specs=[
            pl.BlockSpec((N, D), lambda j: (0, 0)),
            pl.BlockSpec((D, E), lambda j: (0, 0)),
            pl.BlockSpec((1, E), lambda j: (0, 0)),
            pl.BlockSpec((E, BO, D), lambda j: (0, j, 0)),
            pl.BlockSpec((E, BO), lambda j: (0, j)),
        ],
        out_specs=pl.BlockSpec((N, BO), lambda j: (0, j)),
        out_shape=jax.ShapeDtypeStruct((N, O), jnp.float32),
        scratch_shapes=[
            pltpu.VMEM((N, D), jnp.bfloat16),
            pltpu.VMEM((N, E), jnp.bfloat16),
        ],
    )(x, wgt, bg2, We, be)
